# baseline (speedup 1.0000x reference)
import jax
import jax.numpy as jnp
from jax.experimental import pallas as pl
from jax.experimental.pallas import tpu as pltpu

NX, NY, NF = 512, 512, 64
TOPK = 2000
PER = 12000


def _bn(x, g, b, eps=1e-3):
    m = x.mean(axis=(0, 2, 3), keepdims=True)
    v = x.var(axis=(0, 2, 3), keepdims=True)
    return (x - m) / jnp.sqrt(v + eps) * g.reshape(1, -1, 1, 1) + b.reshape(1, -1, 1, 1)


def _conv_im2col(x, w, prec):
    Bn, C, H, W = x.shape
    O = w.shape[0]
    xp = jnp.pad(x, ((0, 0), (0, 0), (1, 1), (1, 1)))
    cols = []
    for ky in range(3):
        for kx in range(3):
            cols.append(jax.lax.slice(xp, (0, 0, ky, kx), (Bn, C, ky + H, kx + W)))
    col = jnp.stack(cols, axis=2)
    col = col.reshape(Bn, C * 9, H * W)
    w2 = w.reshape(O, C * 9)
    out = jnp.einsum('ok,bkn->bon', w2, col, precision=prec)
    return out.reshape(Bn, O, H, W)


def _token_pallas(x):
    def body(x_ref, o_ref):
        o_ref[...] = x_ref[...]
    return pl.pallas_call(
        body, out_shape=jax.ShapeDtypeStruct(x.shape, x.dtype))(x)


def kernel(pillar_features, voxel_cls, W1, g1, b1, W2, g2, b2, voxel_coords):
    P = pillar_features.shape[0]
    batch_size = voxel_coords.shape[0] // PER
    per = P // batch_size
    spatial_list, idx_list = [], []
    for bi in range(batch_size):
        c = voxel_coords[bi * per:(bi + 1) * per]
        ind = c[:, 1] + c[:, 2] * NX + c[:, 3]
        pillars = pillar_features[bi * per:(bi + 1) * per].T
        sp = jnp.zeros((NF, NX * NY), pillar_features.dtype).at[:, ind].set(pillars)
        spatial_list.append(sp)
        idx_list.append(ind)
    batch_sp = jnp.stack(spatial_list, 0).reshape(batch_size, NF, NY, NX)
    h = jax.lax.conv_general_dilated(
        batch_sp, W1, (1, 1), ((1, 1), (1, 1)),
        dimension_numbers=('NCHW', 'OIHW', 'NCHW'))
    h = jax.nn.relu(h)
    h = jax.lax.conv_general_dilated(
        h, W2, (1, 1), ((1, 1), (1, 1)),
        dimension_numbers=('NCHW', 'OIHW', 'NCHW'))
    s = jax.nn.sigmoid(h)

    outs, sels, scores = [], [], []
    for bi in range(batch_size):
        ind = idx_list[bi]
        feats = s[bi].reshape(-1)[ind]
        score, index = jax.lax.top_k(feats, TOPK)
        cls = voxel_cls[bi * per:(bi + 1) * per][index].sum(axis=1, keepdims=True)
        pidx = ind[index]
        sf1 = batch_sp[bi].reshape(NF, -1)
        red = jnp.zeros((NF, NX * NY), pillar_features.dtype).at[:, pidx].set(sf1[:, pidx])
        outs.append(red)
        sels.append(cls)
        scores.append(score)
    out = jnp.stack(outs, 0).reshape(batch_size, NF, NY, NX)
    return out, jnp.stack(sels, 0), _token_pallas(jnp.stack(scores, 0))


# speedup vs baseline: 1.2709x; 1.2709x over previous
import functools

import jax
import jax.numpy as jnp
from jax.experimental import pallas as pl
from jax.experimental.pallas import tpu as pltpu

NX, NY, NF = 512, 512, 64
TOPK = 2000
PER = 12000
NSORT = 16384
LOG2N = 14


def _topk_kernel(feats_ref, val_ref, idx_ref):
    v = feats_ref[...]
    R = jax.lax.broadcasted_iota(jnp.int32, (128, 128), 0)
    C = jax.lax.broadcasted_iota(jnp.int32, (128, 128), 1)
    I = R * 128 + C
    i = I
    v = jnp.where(I >= PER, -jnp.inf, v)
    for s in range(1, LOG2N + 1):
        for t in range(s - 1, -1, -1):
            d = 1 << t
            if d >= 128:
                ax, sh = 0, d // 128
            else:
                ax, sh = 1, d
            v_dn = jnp.roll(v, -sh, axis=ax)
            v_up = jnp.roll(v, sh, axis=ax)
            i_dn = jnp.roll(i, -sh, axis=ax)
            i_up = jnp.roll(i, sh, axis=ax)
            low_m = (I & d) == 0
            lo = ((I & d) ^ d) >> t
            pv = jnp.where(low_m, v_dn, v_up)
            pi = i_dn * lo + i_up * (1 - lo)
            av = jnp.where(low_m, v, pv)
            bv = jnp.where(low_m, pv, v)
            ai = i * lo + pi * (1 - lo)
            bi = pi * lo + i * (1 - lo)
            asc_i = (I >> s) & 1
            lt_i = ((av < bv) | ((av == bv) & (ai > bi))).astype(jnp.int32)
            swap_i = lt_i ^ asc_i
            swap_m = swap_i == 1
            v = jnp.where(swap_m, pv, v)
            i = pi * swap_i + i * (1 - swap_i)
    val_ref[...] = v
    idx_ref[...] = i


def _pallas_topk(feats):
    B = feats.shape[0]
    fpad = jnp.pad(feats, ((0, 0), (0, NSORT - PER))).reshape(B, 128, 128)
    val, idx = pl.pallas_call(
        _topk_kernel,
        grid=(B,),
        in_specs=[pl.BlockSpec((None, 128, 128), lambda b: (b, 0, 0))],
        out_specs=[
            pl.BlockSpec((None, 128, 128), lambda b: (b, 0, 0)),
            pl.BlockSpec((None, 128, 128), lambda b: (b, 0, 0)),
        ],
        out_shape=[
            jax.ShapeDtypeStruct((B, 128, 128), jnp.float32),
            jax.ShapeDtypeStruct((B, 128, 128), jnp.int32),
        ],
    )(fpad)
    return (val.reshape(B, NSORT)[:, :TOPK], idx.reshape(B, NSORT)[:, :TOPK])


def _mask_kernel(pidx_ref, mask_ref):
    mask_ref[...] = jnp.zeros_like(mask_ref)
    lane = jax.lax.broadcasted_iota(jnp.int32, (1, 128), 1)

    def body(k, _):
        c = pidx_ref[0, k]
        r = c // 128
        onehot = (lane == (c % 128)).astype(jnp.float32)
        row = mask_ref[pl.ds(r, 1), :]
        mask_ref[pl.ds(r, 1), :] = jnp.maximum(row, onehot)
        return 0

    jax.lax.fori_loop(0, TOPK, body, 0)


def _pallas_mask(pidx):
    B = pidx.shape[0]
    return pl.pallas_call(
        _mask_kernel,
        grid=(B,),
        in_specs=[pl.BlockSpec((None, 1, TOPK), lambda b: (b, 0, 0),
                               memory_space=pltpu.SMEM)],
        out_specs=pl.BlockSpec((None, NY * NX // 128, 128), lambda b: (b, 0, 0)),
        out_shape=jax.ShapeDtypeStruct((B, NY * NX // 128, 128), jnp.float32),
    )(pidx.reshape(B, 1, TOPK))


def _mul_kernel(x_ref, m_ref, o_ref):
    o_ref[...] = x_ref[...] * m_ref[...]


def _pallas_masked_mul(batch_sp, mask):
    B = batch_sp.shape[0]
    return pl.pallas_call(
        _mul_kernel,
        grid=(B, NF),
        in_specs=[
            pl.BlockSpec((None, None, NY, NX), lambda b, f: (b, f, 0, 0)),
            pl.BlockSpec((None, NY, NX), lambda b, f: (b, 0, 0)),
        ],
        out_specs=pl.BlockSpec((None, None, NY, NX), lambda b, f: (b, f, 0, 0)),
        out_shape=jax.ShapeDtypeStruct(batch_sp.shape, jnp.float32),
    )(batch_sp, mask)


def _conv(x, w):
    return jax.lax.conv_general_dilated(
        x, w, (1, 1), ((1, 1), (1, 1)),
        dimension_numbers=('NCHW', 'OIHW', 'NCHW'))


def _bn(x, g, b, eps=1e-3):
    m = x.mean(axis=(0, 2, 3), keepdims=True)
    v = x.var(axis=(0, 2, 3), keepdims=True)
    return (x - m) / jnp.sqrt(v + eps) * g.reshape(1, -1, 1, 1) + b.reshape(1, -1, 1, 1)


def kernel(pillar_features, voxel_cls, W1, g1, b1, W2, g2, b2, voxel_coords):
    P = pillar_features.shape[0]
    batch_size = voxel_coords.shape[0] // PER
    per = P // batch_size
    spatial_list, idx_list = [], []
    for bi in range(batch_size):
        c = voxel_coords[bi * per:(bi + 1) * per]
        ind = c[:, 1] + c[:, 2] * NX + c[:, 3]
        pillars = pillar_features[bi * per:(bi + 1) * per].T
        sp = jnp.zeros((NF, NX * NY), pillar_features.dtype).at[:, ind].set(pillars)
        spatial_list.append(sp)
        idx_list.append(ind)
    batch_sp = jnp.stack(spatial_list, 0).reshape(batch_size, NF, NY, NX)
    h = _conv(batch_sp, W1)
    h = _bn(h, g1, b1)
    h = jax.nn.relu(h)
    h = _conv(h, W2)
    h = _bn(h, g2, b2)
    s = jax.nn.sigmoid(h)

    feats = jnp.stack(
        [s[bi].reshape(-1)[idx_list[bi]] for bi in range(batch_size)], 0)
    score, index = _pallas_topk(feats)

    sels, pidx_l = [], []
    for bi in range(batch_size):
        cls = voxel_cls[bi * per:(bi + 1) * per][index[bi]].sum(axis=1, keepdims=True)
        sels.append(cls)
        pidx_l.append(idx_list[bi][index[bi]])
    pidx = jnp.stack(pidx_l, 0)
    mask = _pallas_mask(pidx).reshape(batch_size, NY, NX)
    out = _pallas_masked_mul(batch_sp, mask)
    return out, jnp.stack(sels, 0), score
